# Initial kernel scaffold; baseline (speedup 1.0000x reference)
import jax
import jax.numpy as jnp
from jax.experimental import pallas as pl


def kernel(boxes, scores, classes):
    raise NotImplementedError("write your pallas kernel here")


# speedup vs baseline: 189.7060x; 189.7060x over previous
import jax
import jax.numpy as jnp
from jax.experimental import pallas as pl

_SCORE_T = 0.05
_IOU_T = 0.6
_NP = 5120
_BI = 128
_NB = _NP // _BI


def _build_s_kernel(x1c, y1c, x2c, y2c, sc, cc,
                    x1r, y1r, x2r, y2r, sr, cr, s_out):
    i = pl.program_id(0)
    m = jnp.maximum(jnp.maximum(jnp.max(x1r[...]), jnp.max(y1r[...])),
                    jnp.maximum(jnp.max(x2r[...]), jnp.max(y2r[...])))
    scale = m + 1.0
    offr = cr[...] * scale
    offc = cc[...] * scale
    ax1r = x1r[...] + offr
    ay1r = y1r[...] + offr
    ax2r = x2r[...] + offr
    ay2r = y2r[...] + offr
    ax1c = x1c[...] + offc
    ay1c = y1c[...] + offc
    ax2c = x2c[...] + offc
    ay2c = y2c[...] + offc
    areas_r = (ax2r - ax1r + 1.0) * (ay2r - ay1r + 1.0)
    areas_c = (ax2c - ax1c + 1.0) * (ay2c - ay1c + 1.0)
    xmin = jnp.maximum(ax1c, ax1r)
    ymin = jnp.maximum(ay1c, ay1r)
    xmax = jnp.minimum(ax2c, ax2r)
    ymax = jnp.minimum(ay2c, ay2r)
    inter = jnp.maximum(xmax - xmin, 0.0) * jnp.maximum(ymax - ymin, 0.0)
    iou = inter / ((areas_c + areas_r) - inter)
    si = sc[...]
    sj = sr[...]
    idx_i = i * _BI + jax.lax.broadcasted_iota(jnp.int32, (_BI, 1), 0)
    idx_j = jax.lax.broadcasted_iota(jnp.int32, (1, _NP), 1)
    higher = (si > sj) | ((si == sj) & (idx_i < idx_j))
    valid_i = si >= _SCORE_T
    s_out[...] = ((iou > _IOU_T) & higher & valid_i).astype(jnp.int8)


def _sweep_kernel(s_ref, kc_ref, sr_ref, out_ref):
    c = pl.program_id(0)
    nb = pl.num_programs(0)
    part = jnp.max(s_ref[...].astype(jnp.float32) * kc_ref[...],
                   axis=0, keepdims=True)
    prev = jnp.where(c == 0, 0.0, out_ref[...])
    acc = jnp.maximum(prev, part)
    valid = (sr_ref[...] >= _SCORE_T).astype(jnp.float32)
    out_ref[...] = jnp.where(c == nb - 1, valid * (1.0 - acc), acc)


def kernel(boxes, scores, classes):
    n = boxes.shape[0]
    pad = _NP - n
    b = jnp.pad(boxes, ((0, pad), (0, 0)))
    s = jnp.pad(scores, (0, pad), constant_values=-1.0)
    cf = jnp.pad(classes, (0, pad)).astype(boxes.dtype)

    cols = [b[:, k].reshape(_NP, 1) for k in range(4)]
    rows = [b[:, k].reshape(1, _NP) for k in range(4)]
    scol, srow = s.reshape(_NP, 1), s.reshape(1, _NP)
    ccol, crow = cf.reshape(_NP, 1), cf.reshape(1, _NP)

    col_spec = pl.BlockSpec((_BI, 1), lambda i: (i, 0))
    row_spec = pl.BlockSpec((1, _NP), lambda i: (0, 0))
    smat = pl.pallas_call(
        _build_s_kernel,
        grid=(_NB,),
        in_specs=[col_spec] * 6 + [row_spec] * 6,
        out_specs=pl.BlockSpec((_BI, _NP), lambda i: (i, 0)),
        out_shape=jax.ShapeDtypeStruct((_NP, _NP), jnp.int8),
    )(*cols, scol, ccol, *rows, srow, crow)

    sweep_call = pl.pallas_call(
        _sweep_kernel,
        grid=(_NB,),
        in_specs=[
            pl.BlockSpec((_BI, _NP), lambda c: (c, 0)),
            pl.BlockSpec((_BI, 1), lambda c: (c, 0)),
            pl.BlockSpec((1, _NP), lambda c: (0, 0)),
        ],
        out_specs=pl.BlockSpec((1, _NP), lambda c: (0, 0)),
        out_shape=jax.ShapeDtypeStruct((1, _NP), jnp.float32),
    )

    def do_sweep(krow):
        return sweep_call(smat, krow.reshape(_NP, 1), srow)

    k0 = jnp.ones((1, _NP), jnp.float32)
    k1 = do_sweep(k0)

    def cond(st):
        kp, kc, it = st
        return jnp.logical_and(jnp.any(kp != kc), it < _NP + 2)

    def body(st):
        _, kc, it = st
        return (kc, do_sweep(kc), it + 1)

    _, kf, _ = jax.lax.while_loop(cond, body, (k0, k1, jnp.int32(1)))
    mask = kf.reshape(_NP)[:n]
    return jnp.concatenate([(scores * mask)[:, None],
                            boxes * mask[:, None]], axis=-1)


# speedup vs baseline: 273.3883x; 1.4411x over previous
import jax
import jax.numpy as jnp
from jax.experimental import pallas as pl
from jax.experimental.pallas import tpu as pltpu

_SCORE_T = 0.05
_IOU_T = 0.6
_NP = 5120
_BI = 128
_NB = _NP // _BI


def _row_to_col(row_f32):
    ii = jax.lax.broadcasted_iota(jnp.int32, (_BI, _BI), 0)
    jj = jax.lax.broadcasted_iota(jnp.int32, (_BI, _BI), 1)
    eye = (ii == jj).astype(jnp.float32)
    return jnp.max(eye * row_f32, axis=1, keepdims=True)


def _fused_kernel(bc, sc, cc, br, sr, cr, out_ref,
                  s_mat, acc_row, krow, kcol):
    i = pl.program_id(0)

    x1r, y1r = br[0:1, :], br[1:2, :]
    x2r, y2r = br[2:3, :], br[3:4, :]
    m = jnp.maximum(jnp.maximum(jnp.max(x1r), jnp.max(y1r)),
                    jnp.maximum(jnp.max(x2r), jnp.max(y2r)))
    scale = m + 1.0
    offr = cr[...] * scale
    offc = cc[...] * scale
    ax1r, ay1r = x1r + offr, y1r + offr
    ax2r, ay2r = x2r + offr, y2r + offr
    ax1c, ay1c = bc[:, 0:1] + offc, bc[:, 1:2] + offc
    ax2c, ay2c = bc[:, 2:3] + offc, bc[:, 3:4] + offc
    areas_r = (ax2r - ax1r + 1.0) * (ay2r - ay1r + 1.0)
    areas_c = (ax2c - ax1c + 1.0) * (ay2c - ay1c + 1.0)
    dx = jnp.minimum(ax2c, ax2r) - jnp.maximum(ax1c, ax1r)
    dy = jnp.minimum(ay2c, ay2r) - jnp.maximum(ay1c, ay1r)
    inter = jnp.maximum(dx, 0.0) * jnp.maximum(dy, 0.0)
    iou = inter / ((areas_c + areas_r) - inter)
    si = sc[...]
    sj = sr[...]
    si_adj = jnp.where(si >= _SCORE_T, si, -2.0)
    idx_i = i * _BI + jax.lax.broadcasted_iota(jnp.int32, (_BI, 1), 0)
    idx_j = jax.lax.broadcasted_iota(jnp.int32, (1, _NP), 1)
    higher = (si_adj > sj) | ((si_adj == sj) & (idx_i < idx_j))
    smask = (iou > _IOU_T) & higher
    s_mat[pl.ds(i * _BI, _BI), :] = smask.astype(jnp.int8)

    part = jnp.max(jnp.where(smask, 1.0, 0.0),
                   axis=0, keepdims=True)
    prev = jnp.where(i == 0, jnp.zeros_like(part), acc_row[...])
    acc_row[...] = jnp.maximum(prev, part)

    @pl.when(i == _NB - 1)
    def _fixpoint():
        valid_f = (sj >= _SCORE_T).astype(jnp.float32)
        k1 = valid_f * (1.0 - acc_row[...])
        krow[...] = k1

        def set_kcol():
            def chunk(c, carry):
                col = _row_to_col(krow[0:1, pl.ds(c * _BI, _BI)])
                kcol[pl.ds(c * _BI, _BI), :] = col
                return carry
            jax.lax.fori_loop(0, _NB, chunk, 0)

        set_kcol()

        def sweep(changed):
            def chunk(c, sup):
                sb = s_mat[pl.ds(c * _BI, _BI), :]
                kc = kcol[pl.ds(c * _BI, _BI), :]
                hit = jnp.max(sb.astype(jnp.float32) * kc,
                              axis=0, keepdims=True)
                return jnp.maximum(sup, hit)
            sup = jax.lax.fori_loop(
                0, _NB, chunk, jnp.zeros((1, _NP), jnp.float32))
            knew = valid_f * (1.0 - sup)
            changed = jnp.any(knew != krow[...])
            krow[...] = knew
            set_kcol()
            return changed

        jax.lax.while_loop(lambda ch: ch, sweep, jnp.bool_(True))

        kf = krow[...]
        out_ref[0:1, :] = sj * kf
        out_ref[1:5, :] = br[...] * kf


def kernel(boxes, scores, classes):
    n = boxes.shape[0]
    pad = _NP - n
    b = jnp.pad(boxes, ((0, pad), (0, 0)))
    s = jnp.pad(scores, (0, pad), constant_values=-1.0)
    cf = jnp.pad(classes, (0, pad)).astype(boxes.dtype)

    out = pl.pallas_call(
        _fused_kernel,
        grid=(_NB,),
        in_specs=[
            pl.BlockSpec((_BI, 4), lambda i: (i, 0)),
            pl.BlockSpec((_BI, 1), lambda i: (i, 0)),
            pl.BlockSpec((_BI, 1), lambda i: (i, 0)),
            pl.BlockSpec((4, _NP), lambda i: (0, 0)),
            pl.BlockSpec((1, _NP), lambda i: (0, 0)),
            pl.BlockSpec((1, _NP), lambda i: (0, 0)),
        ],
        out_specs=pl.BlockSpec((5, _NP), lambda i: (0, 0)),
        out_shape=jax.ShapeDtypeStruct((5, _NP), jnp.float32),
        scratch_shapes=[
            pltpu.VMEM((_NP, _NP), jnp.int8),
            pltpu.VMEM((1, _NP), jnp.float32),
            pltpu.VMEM((1, _NP), jnp.float32),
            pltpu.VMEM((_NP, 1), jnp.float32),
        ],
    )(b, s.reshape(_NP, 1), cf.reshape(_NP, 1),
      b.T, s.reshape(1, _NP), cf.reshape(1, _NP))

    return out.T[:n]


# speedup vs baseline: 333.4185x; 1.2196x over previous
import jax
import jax.numpy as jnp
from jax.experimental import pallas as pl
from jax.experimental.pallas import tpu as pltpu

_SCORE_T = 0.05
_IOU_T = 0.6
_NP = 5120
_BI = 128
_NB = _NP // _BI


def _row_to_col(row_f32):
    ii = jax.lax.broadcasted_iota(jnp.int32, (_BI, _BI), 0)
    jj = jax.lax.broadcasted_iota(jnp.int32, (_BI, _BI), 1)
    eye = (ii == jj).astype(jnp.float32)
    return jnp.max(eye * row_f32, axis=1, keepdims=True)


def _fused_kernel(bc, sc, cc, br, sr, cr, out_ref,
                  s_mat, acc_row, krow, kcol):
    i = pl.program_id(0)

    x1r, y1r = br[0:1, :], br[1:2, :]
    x2r, y2r = br[2:3, :], br[3:4, :]
    m = jnp.maximum(jnp.maximum(jnp.max(x1r), jnp.max(y1r)),
                    jnp.maximum(jnp.max(x2r), jnp.max(y2r)))
    scale = m + 1.0
    offr = cr[...] * scale
    offc = cc[...] * scale
    ax1r, ay1r = x1r + offr, y1r + offr
    ax2r, ay2r = x2r + offr, y2r + offr
    ax1c, ay1c = bc[:, 0:1] + offc, bc[:, 1:2] + offc
    ax2c, ay2c = bc[:, 2:3] + offc, bc[:, 3:4] + offc
    areas_r = (ax2r - ax1r + 1.0) * (ay2r - ay1r + 1.0)
    areas_c = (ax2c - ax1c + 1.0) * (ay2c - ay1c + 1.0)
    dx = jnp.minimum(ax2c, ax2r) - jnp.maximum(ax1c, ax1r)
    dy = jnp.minimum(ay2c, ay2r) - jnp.maximum(ay1c, ay1r)
    inter = jnp.maximum(dx, 0.0) * jnp.maximum(dy, 0.0)
    iou = inter / ((areas_c + areas_r) - inter)
    si = sc[...]
    sj = sr[...]
    si_adj = jnp.where(si >= _SCORE_T, si, -2.0)

    sjb = jax.lax.bitcast_convert_type(sj, jnp.int32)
    nd = jnp.where(
        sj > 0.0,
        jax.lax.bitcast_convert_type(sjb - 1, jnp.float32),
        jnp.where(sj == 0.0, jnp.float32(-1e-45), sj))
    jj = jax.lax.broadcasted_iota(jnp.int32, (1, _NP), 1)
    sj_mod = jnp.where(jj >= (i + 1) * _BI, nd, sj)
    higher = si_adj > sj_mod
    smask = (iou > _IOU_T) & higher
    s_mat[pl.ds(i * _BI, _BI), :] = smask.astype(jnp.int8)

    part = jnp.max(jnp.where(smask, 1.0, 0.0),
                   axis=0, keepdims=True)
    prev = jnp.where(i == 0, jnp.zeros_like(part), acc_row[...])
    acc_row[...] = jnp.maximum(prev, part)

    dsl = pl.ds(i * _BI, _BI)
    x1rd, y1rd = br[0:1, dsl], br[1:2, dsl]
    x2rd, y2rd = br[2:3, dsl], br[3:4, dsl]
    offrd = cr[0:1, dsl] * scale
    dax1r, day1r = x1rd + offrd, y1rd + offrd
    dax2r, day2r = x2rd + offrd, y2rd + offrd
    dareas_r = (dax2r - dax1r + 1.0) * (day2r - day1r + 1.0)
    ddx = jnp.minimum(ax2c, dax2r) - jnp.maximum(ax1c, dax1r)
    ddy = jnp.minimum(ay2c, day2r) - jnp.maximum(ay1c, day1r)
    dinter = jnp.maximum(ddx, 0.0) * jnp.maximum(ddy, 0.0)
    diou = dinter / ((areas_c + dareas_r) - dinter)
    dsj = sr[0:1, dsl]
    idx_i = i * _BI + jax.lax.broadcasted_iota(jnp.int32, (_BI, 1), 0)
    didx_j = i * _BI + jax.lax.broadcasted_iota(jnp.int32, (1, _BI), 1)
    dhigher = (si_adj > dsj) | ((si_adj == dsj) & (idx_i < didx_j))
    dsmask = (diou > _IOU_T) & dhigher
    s_mat[pl.ds(i * _BI, _BI), pl.ds(i * _BI, _BI)] = (
        dsmask.astype(jnp.int8))

    @pl.when(i == _NB - 1)
    def _fixpoint():
        valid_f = (sj >= _SCORE_T).astype(jnp.float32)
        k1 = valid_f * (1.0 - acc_row[...])
        krow[...] = k1

        def set_kcol():
            def chunk(c, carry):
                col = _row_to_col(krow[0:1, pl.ds(c * _BI, _BI)])
                kcol[pl.ds(c * _BI, _BI), :] = col.astype(jnp.bfloat16)
                return carry
            jax.lax.fori_loop(0, _NB, chunk, 0)

        set_kcol()

        def sweep(changed):
            def chunk(c, sup):
                sb = s_mat[pl.ds(c * _BI, _BI), :]
                kc = kcol[pl.ds(c * _BI, _BI), :]
                hit = jnp.max(sb.astype(jnp.bfloat16) * kc,
                              axis=0, keepdims=True)
                return jnp.maximum(sup, hit)
            sup = jax.lax.fori_loop(
                0, _NB, chunk, jnp.zeros((1, _NP), jnp.bfloat16))
            knew = valid_f * (1.0 - sup.astype(jnp.float32))
            changed = jnp.any(knew != krow[...])
            krow[...] = knew
            set_kcol()
            return changed

        jax.lax.while_loop(lambda ch: ch, sweep, jnp.bool_(True))

        kf = krow[...]
        out_ref[0:1, :] = sj * kf
        out_ref[1:5, :] = br[...] * kf


def kernel(boxes, scores, classes):
    n = boxes.shape[0]
    pad = _NP - n
    b = jnp.pad(boxes, ((0, pad), (0, 0)))
    s = jnp.pad(scores, (0, pad), constant_values=-1.0)
    cf = jnp.pad(classes, (0, pad)).astype(boxes.dtype)

    out = pl.pallas_call(
        _fused_kernel,
        grid=(_NB,),
        in_specs=[
            pl.BlockSpec((_BI, 4), lambda i: (i, 0)),
            pl.BlockSpec((_BI, 1), lambda i: (i, 0)),
            pl.BlockSpec((_BI, 1), lambda i: (i, 0)),
            pl.BlockSpec((4, _NP), lambda i: (0, 0)),
            pl.BlockSpec((1, _NP), lambda i: (0, 0)),
            pl.BlockSpec((1, _NP), lambda i: (0, 0)),
        ],
        out_specs=pl.BlockSpec((5, _NP), lambda i: (0, 0)),
        out_shape=jax.ShapeDtypeStruct((5, _NP), jnp.float32),
        scratch_shapes=[
            pltpu.VMEM((_NP, _NP), jnp.int8),
            pltpu.VMEM((1, _NP), jnp.float32),
            pltpu.VMEM((1, _NP), jnp.float32),
            pltpu.VMEM((_NP, 1), jnp.bfloat16),
        ],
    )(b, s.reshape(_NP, 1), cf.reshape(_NP, 1),
      b.T, s.reshape(1, _NP), cf.reshape(1, _NP))

    return out.T[:n]


# speedup vs baseline: 340.4878x; 1.0212x over previous
import jax
import jax.numpy as jnp
from jax.experimental import pallas as pl
from jax.experimental.pallas import tpu as pltpu

_SCORE_T = 0.05
_IOU_T = 0.6
_NP = 5120
_BI = 128
_NB = _NP // _BI


def _row_to_col(row_f32):
    ii = jax.lax.broadcasted_iota(jnp.int32, (_BI, _BI), 0)
    jj = jax.lax.broadcasted_iota(jnp.int32, (_BI, _BI), 1)
    eye = (ii == jj).astype(jnp.float32)
    return jnp.max(eye * row_f32, axis=1, keepdims=True)


def _fused_kernel(bc, sc, cc, br, sr, cr, out_ref,
                  s_mat, prep, scale_sm, krow, kcol):
    i = pl.program_id(0)

    @pl.when(i == 0)
    def _prep():
        x1r, y1r = br[0:1, :], br[1:2, :]
        x2r, y2r = br[2:3, :], br[3:4, :]
        m = jnp.maximum(jnp.maximum(jnp.max(x1r), jnp.max(y1r)),
                        jnp.maximum(jnp.max(x2r), jnp.max(y2r)))
        scale_sm[0] = m + 1.0
        offr = cr[...] * (m + 1.0)
        prep[0:1, :] = x1r + offr
        prep[1:2, :] = y1r + offr
        prep[2:3, :] = x2r + offr
        prep[3:4, :] = y2r + offr
        prep[4:5, :] = ((prep[2:3, :] - prep[0:1, :] + 1.0)
                        * (prep[3:4, :] - prep[1:2, :] + 1.0))
        sj0 = sr[...]
        sjb = jax.lax.bitcast_convert_type(sj0, jnp.int32)
        prep[5:6, :] = jnp.where(
            sj0 > 0.0,
            jax.lax.bitcast_convert_type(sjb - 1, jnp.float32),
            jnp.where(sj0 == 0.0, jnp.float32(-1e-45), sj0))

    scale = scale_sm[0]
    ax1r, ay1r = prep[0:1, :], prep[1:2, :]
    ax2r, ay2r = prep[2:3, :], prep[3:4, :]
    areas_r = prep[4:5, :]
    nd = prep[5:6, :]
    sj = sr[...]

    offc = cc[...] * scale
    ax1c, ay1c = bc[:, 0:1] + offc, bc[:, 1:2] + offc
    ax2c, ay2c = bc[:, 2:3] + offc, bc[:, 3:4] + offc
    areas_c = (ax2c - ax1c + 1.0) * (ay2c - ay1c + 1.0)
    dx = jnp.minimum(ax2c, ax2r) - jnp.maximum(ax1c, ax1r)
    dy = jnp.minimum(ay2c, ay2r) - jnp.maximum(ay1c, ay1r)
    inter = jnp.maximum(dx, 0.0) * jnp.maximum(dy, 0.0)
    iou = inter / ((areas_c + areas_r) - inter)
    si = sc[...]
    si_adj = jnp.where(si >= _SCORE_T, si, -2.0)
    jj = jax.lax.broadcasted_iota(jnp.int32, (1, _NP), 1)
    sj_mod = jnp.where(jj >= (i + 1) * _BI, nd, sj)
    higher = si_adj > sj_mod
    smask = (iou > _IOU_T) & higher
    s_mat[pl.ds(i * _BI, _BI), :] = smask.astype(jnp.int8)

    dsl = pl.ds(i * _BI, _BI)
    dax1r, day1r = prep[0:1, dsl], prep[1:2, dsl]
    dax2r, day2r = prep[2:3, dsl], prep[3:4, dsl]
    dareas_r = prep[4:5, dsl]
    ddx = jnp.minimum(ax2c, dax2r) - jnp.maximum(ax1c, dax1r)
    ddy = jnp.minimum(ay2c, day2r) - jnp.maximum(ay1c, day1r)
    dinter = jnp.maximum(ddx, 0.0) * jnp.maximum(ddy, 0.0)
    diou = dinter / ((areas_c + dareas_r) - dinter)
    dsj = sr[0:1, dsl]
    idx_i = i * _BI + jax.lax.broadcasted_iota(jnp.int32, (_BI, 1), 0)
    didx_j = i * _BI + jax.lax.broadcasted_iota(jnp.int32, (1, _BI), 1)
    dhigher = (si_adj > dsj) | ((si_adj == dsj) & (idx_i < didx_j))
    dsmask = (diou > _IOU_T) & dhigher
    s_mat[pl.ds(i * _BI, _BI), pl.ds(i * _BI, _BI)] = (
        dsmask.astype(jnp.int8))

    @pl.when(i == _NB - 1)
    def _fixpoint():
        valid_f = (sj >= _SCORE_T).astype(jnp.float32)
        krow[...] = valid_f

        def set_kcol():
            def chunk(c, carry):
                col = _row_to_col(krow[0:1, pl.ds(c * _BI, _BI)])
                kcol[pl.ds(c * _BI, _BI), :] = col.astype(jnp.bfloat16)
                return carry
            jax.lax.fori_loop(0, _NB, chunk, 0)

        set_kcol()

        def sweep(changed):
            def chunk(c, sup):
                sb = s_mat[pl.ds(c * _BI, _BI), :]
                kc = kcol[pl.ds(c * _BI, _BI), :]
                hit = jnp.max(sb.astype(jnp.bfloat16) * kc,
                              axis=0, keepdims=True)
                return jnp.maximum(sup, hit)
            sup = jax.lax.fori_loop(
                0, _NB, chunk, jnp.zeros((1, _NP), jnp.bfloat16))
            knew = valid_f * (1.0 - sup.astype(jnp.float32))
            changed = jnp.any(knew != krow[...])
            krow[...] = knew
            set_kcol()
            return changed

        jax.lax.while_loop(lambda ch: ch, sweep, jnp.bool_(True))

        kf = krow[...]
        out_ref[0:1, :] = sj * kf
        out_ref[1:5, :] = br[...] * kf


def kernel(boxes, scores, classes):
    n = boxes.shape[0]
    pad = _NP - n
    b = jnp.pad(boxes, ((0, pad), (0, 0)))
    s = jnp.pad(scores, (0, pad), constant_values=-1.0)
    cf = jnp.pad(classes, (0, pad)).astype(boxes.dtype)

    out = pl.pallas_call(
        _fused_kernel,
        grid=(_NB,),
        in_specs=[
            pl.BlockSpec((_BI, 4), lambda i: (i, 0)),
            pl.BlockSpec((_BI, 1), lambda i: (i, 0)),
            pl.BlockSpec((_BI, 1), lambda i: (i, 0)),
            pl.BlockSpec((4, _NP), lambda i: (0, 0)),
            pl.BlockSpec((1, _NP), lambda i: (0, 0)),
            pl.BlockSpec((1, _NP), lambda i: (0, 0)),
        ],
        out_specs=pl.BlockSpec((5, _NP), lambda i: (0, 0)),
        out_shape=jax.ShapeDtypeStruct((5, _NP), jnp.float32),
        scratch_shapes=[
            pltpu.VMEM((_NP, _NP), jnp.int8),
            pltpu.VMEM((6, _NP), jnp.float32),
            pltpu.SMEM((1,), jnp.float32),
            pltpu.VMEM((1, _NP), jnp.float32),
            pltpu.VMEM((_NP, 1), jnp.bfloat16),
        ],
    )(b, s.reshape(_NP, 1), cf.reshape(_NP, 1),
      b.T, s.reshape(1, _NP), cf.reshape(1, _NP))

    return out.T[:n]


# speedup vs baseline: 361.8296x; 1.0627x over previous
import jax
import jax.numpy as jnp
from jax.experimental import pallas as pl
from jax.experimental.pallas import tpu as pltpu

_SCORE_T = 0.05
_IOU_T = 0.6
_NP = 5120
_BI = 256
_NB = _NP // _BI


def _row_to_col(row_f32):
    ii = jax.lax.broadcasted_iota(jnp.int32, (_BI, _BI), 0)
    jj = jax.lax.broadcasted_iota(jnp.int32, (_BI, _BI), 1)
    eye = (ii == jj).astype(jnp.float32)
    return jnp.max(eye * row_f32, axis=1, keepdims=True)


def _fused_kernel(bc, sc, cc, br, sr, cr, out_ref,
                  s_mat, prep, scale_sm, krow, kcol):
    i = pl.program_id(0)

    @pl.when(i == 0)
    def _prep():
        x1r, y1r = br[0:1, :], br[1:2, :]
        x2r, y2r = br[2:3, :], br[3:4, :]
        m = jnp.maximum(jnp.maximum(jnp.max(x1r), jnp.max(y1r)),
                        jnp.maximum(jnp.max(x2r), jnp.max(y2r)))
        scale_sm[0] = m + 1.0
        offr = cr[...] * (m + 1.0)
        prep[0:1, :] = x1r + offr
        prep[1:2, :] = y1r + offr
        prep[2:3, :] = x2r + offr
        prep[3:4, :] = y2r + offr
        prep[4:5, :] = ((prep[2:3, :] - prep[0:1, :] + 1.0)
                        * (prep[3:4, :] - prep[1:2, :] + 1.0))
        sj0 = sr[...]
        sjb = jax.lax.bitcast_convert_type(sj0, jnp.int32)
        prep[5:6, :] = jnp.where(
            sj0 > 0.0,
            jax.lax.bitcast_convert_type(sjb - 1, jnp.float32),
            jnp.where(sj0 == 0.0, jnp.float32(-1e-45), sj0))

    scale = scale_sm[0]
    ax1r, ay1r = prep[0:1, :], prep[1:2, :]
    ax2r, ay2r = prep[2:3, :], prep[3:4, :]
    areas_r = prep[4:5, :]
    nd = prep[5:6, :]
    sj = sr[...]

    offc = cc[...] * scale
    ax1c, ay1c = bc[:, 0:1] + offc, bc[:, 1:2] + offc
    ax2c, ay2c = bc[:, 2:3] + offc, bc[:, 3:4] + offc
    areas_c = (ax2c - ax1c + 1.0) * (ay2c - ay1c + 1.0)
    dx = jnp.minimum(ax2c, ax2r) - jnp.maximum(ax1c, ax1r)
    dy = jnp.minimum(ay2c, ay2r) - jnp.maximum(ay1c, ay1r)
    inter = jnp.maximum(dx, 0.0) * jnp.maximum(dy, 0.0)
    iou = inter / ((areas_c + areas_r) - inter)
    si = sc[...]
    si_adj = jnp.where(si >= _SCORE_T, si, -2.0)
    jj = jax.lax.broadcasted_iota(jnp.int32, (1, _NP), 1)
    sj_mod = jnp.where(jj >= (i + 1) * _BI, nd, sj)
    higher = si_adj > sj_mod
    smask = (iou > _IOU_T) & higher
    s_mat[pl.ds(i * _BI, _BI), :] = smask.astype(jnp.int8)

    dsl = pl.ds(i * _BI, _BI)
    dax1r, day1r = prep[0:1, dsl], prep[1:2, dsl]
    dax2r, day2r = prep[2:3, dsl], prep[3:4, dsl]
    dareas_r = prep[4:5, dsl]
    ddx = jnp.minimum(ax2c, dax2r) - jnp.maximum(ax1c, dax1r)
    ddy = jnp.minimum(ay2c, day2r) - jnp.maximum(ay1c, day1r)
    dinter = jnp.maximum(ddx, 0.0) * jnp.maximum(ddy, 0.0)
    diou = dinter / ((areas_c + dareas_r) - dinter)
    dsj = sr[0:1, dsl]
    idx_i = i * _BI + jax.lax.broadcasted_iota(jnp.int32, (_BI, 1), 0)
    didx_j = i * _BI + jax.lax.broadcasted_iota(jnp.int32, (1, _BI), 1)
    dhigher = (si_adj > dsj) | ((si_adj == dsj) & (idx_i < didx_j))
    dsmask = (diou > _IOU_T) & dhigher
    s_mat[pl.ds(i * _BI, _BI), pl.ds(i * _BI, _BI)] = (
        dsmask.astype(jnp.int8))

    @pl.when(i == _NB - 1)
    def _fixpoint():
        valid_f = (sj >= _SCORE_T).astype(jnp.float32)
        krow[...] = valid_f

        def set_kcol():
            def chunk(c, carry):
                col = _row_to_col(krow[0:1, pl.ds(c * _BI, _BI)])
                kcol[pl.ds(c * _BI, _BI), :] = col.astype(jnp.bfloat16)
                return carry
            jax.lax.fori_loop(0, _NB, chunk, 0)

        set_kcol()

        def sweep(changed):
            def chunk(c, sup):
                sb = s_mat[pl.ds(c * _BI, _BI), :]
                kc = kcol[pl.ds(c * _BI, _BI), :]
                hit = jnp.max(sb.astype(jnp.bfloat16) * kc,
                              axis=0, keepdims=True)
                return jnp.maximum(sup, hit)
            sup = jax.lax.fori_loop(
                0, _NB, chunk, jnp.zeros((1, _NP), jnp.bfloat16))
            knew = valid_f * (1.0 - sup.astype(jnp.float32))
            changed = jnp.any(knew != krow[...])
            krow[...] = knew
            set_kcol()
            return changed

        jax.lax.while_loop(lambda ch: ch, sweep, jnp.bool_(True))

        kf = krow[...]
        out_ref[0:1, :] = sj * kf
        out_ref[1:5, :] = br[...] * kf


def kernel(boxes, scores, classes):
    n = boxes.shape[0]
    pad = _NP - n
    b = jnp.pad(boxes, ((0, pad), (0, 0)))
    s = jnp.pad(scores, (0, pad), constant_values=-1.0)
    cf = jnp.pad(classes, (0, pad)).astype(boxes.dtype)

    out = pl.pallas_call(
        _fused_kernel,
        grid=(_NB,),
        in_specs=[
            pl.BlockSpec((_BI, 4), lambda i: (i, 0)),
            pl.BlockSpec((_BI, 1), lambda i: (i, 0)),
            pl.BlockSpec((_BI, 1), lambda i: (i, 0)),
            pl.BlockSpec((4, _NP), lambda i: (0, 0)),
            pl.BlockSpec((1, _NP), lambda i: (0, 0)),
            pl.BlockSpec((1, _NP), lambda i: (0, 0)),
        ],
        out_specs=pl.BlockSpec((5, _NP), lambda i: (0, 0)),
        out_shape=jax.ShapeDtypeStruct((5, _NP), jnp.float32),
        scratch_shapes=[
            pltpu.VMEM((_NP, _NP), jnp.int8),
            pltpu.VMEM((6, _NP), jnp.float32),
            pltpu.SMEM((1,), jnp.float32),
            pltpu.VMEM((1, _NP), jnp.float32),
            pltpu.VMEM((_NP, 1), jnp.bfloat16),
        ],
    )(b, s.reshape(_NP, 1), cf.reshape(_NP, 1),
      b.T, s.reshape(1, _NP), cf.reshape(1, _NP))

    return out.T[:n]


# speedup vs baseline: 369.1347x; 1.0202x over previous
import jax
import jax.numpy as jnp
from jax.experimental import pallas as pl
from jax.experimental.pallas import tpu as pltpu

_SCORE_T = 0.05
_IOU_T = 0.6
_NP = 5120
_BI = 512
_NB = _NP // _BI


def _row_to_col(row_f32):
    ii = jax.lax.broadcasted_iota(jnp.int32, (_BI, _BI), 0)
    jj = jax.lax.broadcasted_iota(jnp.int32, (_BI, _BI), 1)
    eye = (ii == jj).astype(jnp.float32)
    return jnp.max(eye * row_f32, axis=1, keepdims=True)


def _fused_kernel(bc, sc, cc, br, sr, cr, out_ref,
                  s_mat, prep, scale_sm, krow, kcol):
    i = pl.program_id(0)

    @pl.when(i == 0)
    def _prep():
        x1r, y1r = br[0:1, :], br[1:2, :]
        x2r, y2r = br[2:3, :], br[3:4, :]
        m = jnp.maximum(jnp.maximum(jnp.max(x1r), jnp.max(y1r)),
                        jnp.maximum(jnp.max(x2r), jnp.max(y2r)))
        scale_sm[0] = m + 1.0
        offr = cr[...] * (m + 1.0)
        prep[0:1, :] = x1r + offr
        prep[1:2, :] = y1r + offr
        prep[2:3, :] = x2r + offr
        prep[3:4, :] = y2r + offr
        prep[4:5, :] = ((prep[2:3, :] - prep[0:1, :] + 1.0)
                        * (prep[3:4, :] - prep[1:2, :] + 1.0))
        sj0 = sr[...]
        sjb = jax.lax.bitcast_convert_type(sj0, jnp.int32)
        prep[5:6, :] = jnp.where(
            sj0 > 0.0,
            jax.lax.bitcast_convert_type(sjb - 1, jnp.float32),
            jnp.where(sj0 == 0.0, jnp.float32(-1e-45), sj0))

    scale = scale_sm[0]
    ax1r, ay1r = prep[0:1, :], prep[1:2, :]
    ax2r, ay2r = prep[2:3, :], prep[3:4, :]
    areas_r = prep[4:5, :]
    nd = prep[5:6, :]
    sj = sr[...]

    offc = cc[...] * scale
    ax1c, ay1c = bc[:, 0:1] + offc, bc[:, 1:2] + offc
    ax2c, ay2c = bc[:, 2:3] + offc, bc[:, 3:4] + offc
    areas_c = (ax2c - ax1c + 1.0) * (ay2c - ay1c + 1.0)
    dx = jnp.minimum(ax2c, ax2r) - jnp.maximum(ax1c, ax1r)
    dy = jnp.minimum(ay2c, ay2r) - jnp.maximum(ay1c, ay1r)
    inter = jnp.maximum(dx, 0.0) * jnp.maximum(dy, 0.0)
    iou = inter / ((areas_c + areas_r) - inter)
    si = sc[...]
    si_adj = jnp.where(si >= _SCORE_T, si, -2.0)
    jj = jax.lax.broadcasted_iota(jnp.int32, (1, _NP), 1)
    sj_mod = jnp.where(jj >= (i + 1) * _BI, nd, sj)
    higher = si_adj > sj_mod
    smask = (iou > _IOU_T) & higher
    s_mat[pl.ds(i * _BI, _BI), :] = smask.astype(jnp.int8)

    dsl = pl.ds(i * _BI, _BI)
    dax1r, day1r = prep[0:1, dsl], prep[1:2, dsl]
    dax2r, day2r = prep[2:3, dsl], prep[3:4, dsl]
    dareas_r = prep[4:5, dsl]
    ddx = jnp.minimum(ax2c, dax2r) - jnp.maximum(ax1c, dax1r)
    ddy = jnp.minimum(ay2c, day2r) - jnp.maximum(ay1c, day1r)
    dinter = jnp.maximum(ddx, 0.0) * jnp.maximum(ddy, 0.0)
    diou = dinter / ((areas_c + dareas_r) - dinter)
    dsj = sr[0:1, dsl]
    idx_i = i * _BI + jax.lax.broadcasted_iota(jnp.int32, (_BI, 1), 0)
    didx_j = i * _BI + jax.lax.broadcasted_iota(jnp.int32, (1, _BI), 1)
    dhigher = (si_adj > dsj) | ((si_adj == dsj) & (idx_i < didx_j))
    dsmask = (diou > _IOU_T) & dhigher
    s_mat[pl.ds(i * _BI, _BI), pl.ds(i * _BI, _BI)] = (
        dsmask.astype(jnp.int8))

    @pl.when(i == _NB - 1)
    def _fixpoint():
        valid_f = (sj >= _SCORE_T).astype(jnp.float32)
        krow[...] = valid_f

        def set_kcol():
            def chunk(c, carry):
                col = _row_to_col(krow[0:1, pl.ds(c * _BI, _BI)])
                kcol[pl.ds(c * _BI, _BI), :] = col.astype(jnp.bfloat16)
                return carry
            jax.lax.fori_loop(0, _NB, chunk, 0)

        set_kcol()

        def sweep(changed):
            def chunk(c, sup):
                sb = s_mat[pl.ds(c * _BI, _BI), :]
                kc = kcol[pl.ds(c * _BI, _BI), :]
                hit = jnp.max(sb.astype(jnp.bfloat16) * kc,
                              axis=0, keepdims=True)
                return jnp.maximum(sup, hit)
            sup = jax.lax.fori_loop(
                0, _NB, chunk, jnp.zeros((1, _NP), jnp.bfloat16))
            knew = valid_f * (1.0 - sup.astype(jnp.float32))
            changed = jnp.any(knew != krow[...])
            krow[...] = knew
            set_kcol()
            return changed

        jax.lax.while_loop(lambda ch: ch, sweep, jnp.bool_(True))

        kf = krow[...]
        out_ref[0:1, :] = sj * kf
        out_ref[1:5, :] = br[...] * kf


def kernel(boxes, scores, classes):
    n = boxes.shape[0]
    pad = _NP - n
    b = jnp.pad(boxes, ((0, pad), (0, 0)))
    s = jnp.pad(scores, (0, pad), constant_values=-1.0)
    cf = jnp.pad(classes, (0, pad)).astype(boxes.dtype)

    out = pl.pallas_call(
        _fused_kernel,
        grid=(_NB,),
        in_specs=[
            pl.BlockSpec((_BI, 4), lambda i: (i, 0)),
            pl.BlockSpec((_BI, 1), lambda i: (i, 0)),
            pl.BlockSpec((_BI, 1), lambda i: (i, 0)),
            pl.BlockSpec((4, _NP), lambda i: (0, 0)),
            pl.BlockSpec((1, _NP), lambda i: (0, 0)),
            pl.BlockSpec((1, _NP), lambda i: (0, 0)),
        ],
        out_specs=pl.BlockSpec((5, _NP), lambda i: (0, 0)),
        out_shape=jax.ShapeDtypeStruct((5, _NP), jnp.float32),
        scratch_shapes=[
            pltpu.VMEM((_NP, _NP), jnp.int8),
            pltpu.VMEM((6, _NP), jnp.float32),
            pltpu.SMEM((1,), jnp.float32),
            pltpu.VMEM((1, _NP), jnp.float32),
            pltpu.VMEM((_NP, 1), jnp.bfloat16),
        ],
    )(b, s.reshape(_NP, 1), cf.reshape(_NP, 1),
      b.T, s.reshape(1, _NP), cf.reshape(1, _NP))

    return out.T[:n]
